# Initial kernel scaffold; baseline (speedup 1.0000x reference)
import jax
import jax.numpy as jnp
from jax import lax
from jax.experimental import pallas as pl
from jax.experimental.pallas import tpu as pltpu
from jax.experimental.pallas import tpu_sc as plsc

B, T, N = 1, 8, 10000
FD, FS, FE = 8, 16, 4
H, HEADS, TOUT = 64, 4, 6
C = H // HEADS
E = 160000

NCORE, NSUB = 2, 16
CHUNK = 128
NCHUNK = (E + NSUB * CHUNK - 1) // (NSUB * CHUNK)
EPT = NCHUNK * CHUNK
E_PAD = NSUB * EPT
N_PAD = 10016
RPT = N_PAD // NSUB
ROW = 80
TPC = T // NCORE
NBLK = 1000
EBLK = 2048
TOUT_PAD = 8

_HI = jax.lax.Precision.HIGHEST


def _a1_body(x_ref, na_ref, w1x_ref, w1s_ref, b1_ref, wg_ref, asrc_ref,
             adst_ref, hf_ref, xp_ref, als_ref, ald_ref):
    x = x_ref[0]
    na = na_ref[...]
    h = jnp.dot(x, w1x_ref[...], precision=_HI)
    h = h + jnp.dot(na, w1s_ref[...], precision=_HI) + b1_ref[0]
    h = jnp.maximum(h, 0.0)
    xp = jnp.dot(h, wg_ref[...], precision=_HI)
    hf_ref[0] = h
    xp_ref[0] = xp
    als_ref[0] = jnp.dot(xp, asrc_ref[...], precision=_HI)
    ald_ref[0] = jnp.dot(xp, adst_ref[...], precision=_HI)


def _tc_a1(x_seq3, node_attr, w1x, w1s, b1r, Wg, Asrc, Adst):
    grid = (T, N // NBLK)
    out_shape = (
        jax.ShapeDtypeStruct((T, N, H), jnp.float32),
        jax.ShapeDtypeStruct((T, N, H), jnp.float32),
        jax.ShapeDtypeStruct((T, N, HEADS), jnp.float32),
        jax.ShapeDtypeStruct((T, N, HEADS), jnp.float32),
    )
    return pl.pallas_call(
        _a1_body,
        grid=grid,
        in_specs=[
            pl.BlockSpec((1, NBLK, FD), lambda t, i: (t, i, 0)),
            pl.BlockSpec((NBLK, FS), lambda t, i: (i, 0)),
            pl.BlockSpec((FD, H), lambda t, i: (0, 0)),
            pl.BlockSpec((FS, H), lambda t, i: (0, 0)),
            pl.BlockSpec((1, H), lambda t, i: (0, 0)),
            pl.BlockSpec((H, H), lambda t, i: (0, 0)),
            pl.BlockSpec((H, HEADS), lambda t, i: (0, 0)),
            pl.BlockSpec((H, HEADS), lambda t, i: (0, 0)),
        ],
        out_specs=(
            pl.BlockSpec((1, NBLK, H), lambda t, i: (t, i, 0)),
            pl.BlockSpec((1, NBLK, H), lambda t, i: (t, i, 0)),
            pl.BlockSpec((1, NBLK, HEADS), lambda t, i: (t, i, 0)),
            pl.BlockSpec((1, NBLK, HEADS), lambda t, i: (t, i, 0)),
        ),
        out_shape=out_shape,
    )(x_seq3, node_attr, w1x, w1s, b1r, Wg, Asrc, Adst)


def _a2_body(ea_ref, weve_ref, beve_ref, out_ref):
    out_ref[...] = (
        jnp.dot(ea_ref[...], weve_ref[...], precision=_HI) + beve_ref[0]
    )


def _tc_a2(ea_pad, WeVe, beVe_r):
    return pl.pallas_call(
        _a2_body,
        grid=(E_PAD // EBLK,),
        in_specs=[
            pl.BlockSpec((EBLK, FE), lambda i: (i, 0)),
            pl.BlockSpec((FE, HEADS), lambda i: (0, 0)),
            pl.BlockSpec((1, HEADS), lambda i: (0, 0)),
        ],
        out_specs=pl.BlockSpec((EBLK, HEADS), lambda i: (i, 0)),
        out_shape=jax.ShapeDtypeStruct((E_PAD, HEADS), jnp.float32),
    )(ea_pad, WeVe, beVe_r)


def _sc_body(src_h, dst_h, ae_h, asrc_h, adst_h, xp_h,
             nd_out, deg_out,
             asrc_v, adst_v, src_c, dst_c, idx_c, ae_c, xg, msg, row16,
             zrow, nd_sp, deg_sp, sem):
    cid = lax.axis_index("c")
    sid = lax.axis_index("s")
    ebase = sid * EPT
    iota = lax.iota(jnp.int32, 16)
    zeros16 = jnp.zeros((16,), jnp.float32)
    ones16 = jnp.ones((16,), jnp.float32)

    def _zrow_init(i, carry):
        r = i // 5
        c = (i % 5) * 16
        zrow[r, pl.ds(c, 16)] = zeros16
        return carry
    lax.fori_loop(0, 64 * 5, _zrow_init, 0)

    def _row16_init(i, carry):
        row16[i, pl.ds(0, 16)] = zeros16
        return carry
    lax.fori_loop(0, CHUNK, _row16_init, 0)

    pltpu.sync_copy(zrow.at[pl.ds(0, 64)], msg.at[pl.ds(0, 64)])
    pltpu.sync_copy(zrow.at[pl.ds(0, 64)], msg.at[pl.ds(64, 64)])

    for k in range(4):
        asrc_v[pl.ds(N * 4 + k * 16, 16)] = zeros16
        adst_v[pl.ds(N * 4 + k * 16, 16)] = zeros16

    @pl.when(cid == 0)
    def _phase0():
        for k in range(5):
            sz = 128 if k < 4 else RPT - 4 * 128
            pltpu.sync_copy(row16.at[pl.ds(0, sz)],
                            deg_sp.at[pl.ds(sid * RPT + k * 128, sz)])
        plsc.subcore_barrier()

        def _p0_chunk(j, carry):
            base = ebase + j * CHUNK
            pltpu.sync_copy(dst_h.at[pl.ds(base, CHUNK)], dst_c)
            pltpu.sync_copy(ae_h.at[pl.ds(base * 4, CHUNK * 4)], ae_c)
            for g in range(8):
                k16 = iota + g * 16
                plsc.store_scatter(row16, [k16, jnp.full((16,), 0, jnp.int32)],
                                   ones16)
                for h in range(HEADS):
                    ae = plsc.load_gather(ae_c, [k16 * 4 + h])
                    plsc.store_scatter(
                        row16, [k16, jnp.full((16,), 1 + h, jnp.int32)], ae)
            pltpu.sync_copy(row16, deg_sp.at[dst_c], add=True)
            return carry
        lax.fori_loop(0, NCHUNK, _p0_chunk, 0)
        plsc.subcore_barrier()
        for k in range(5):
            sz = 128 if k < 4 else RPT - 4 * 128
            r0 = sid * RPT + k * 128
            pltpu.sync_copy(deg_sp.at[pl.ds(r0, sz)],
                            deg_out.at[pl.ds(r0, sz)])

    def _t_step(jt, carry):
        t = cid * TPC + jt
        for k in range(10):
            sz = 64 if k < 9 else RPT - 9 * 64
            pltpu.sync_copy(zrow.at[pl.ds(0, sz)],
                            nd_sp.at[pl.ds(sid * RPT + k * 64, sz)])
        pltpu.sync_copy(asrc_h.at[t], asrc_v.at[pl.ds(0, N * 4)])
        pltpu.sync_copy(adst_h.at[t], adst_v.at[pl.ds(0, N * 4)])
        plsc.subcore_barrier()

        tN = t * N

        def _chunk(j, carry2):
            base = ebase + j * CHUNK
            pltpu.sync_copy(src_h.at[pl.ds(base, CHUNK)], src_c)
            pltpu.sync_copy(dst_h.at[pl.ds(base, CHUNK)], dst_c)
            pltpu.sync_copy(ae_h.at[pl.ds(base * 4, CHUNK * 4)], ae_c)
            for g in range(8):
                s = src_c[pl.ds(g * 16, 16)]
                idx_c[pl.ds(g * 16, 16)] = s + tN
            pltpu.async_copy(xp_h.at[idx_c], xg, sem).wait()
            for g in range(8):
                k16 = iota + g * 16
                vsrc = src_c[pl.ds(g * 16, 16)]
                vdst = dst_c[pl.ds(g * 16, 16)]
                exs = []
                for h in range(HEADS):
                    a = (plsc.load_gather(asrc_v, [vsrc * 4 + h])
                         + plsc.load_gather(adst_v, [vdst * 4 + h])
                         + plsc.load_gather(ae_c, [k16 * 4 + h]))
                    a = jnp.where(a >= 0.0, a, 0.2 * a)
                    ex = jnp.exp(a)
                    plsc.store_scatter(
                        msg, [k16, jnp.full((16,), H + h, jnp.int32)], ex)
                    exs.append(ex)
                for col in range(H):
                    v = plsc.load_gather(xg, [k16, jnp.full((16,), col,
                                                            jnp.int32)])
                    plsc.store_scatter(
                        msg, [k16, jnp.full((16,), col, jnp.int32)],
                        v * exs[col // 16])
            pltpu.sync_copy(msg, nd_sp.at[dst_c], add=True)
            return carry2
        lax.fori_loop(0, NCHUNK, _chunk, 0)
        plsc.subcore_barrier()
        for k in range(10):
            sz = 64 if k < 9 else RPT - 9 * 64
            r0 = sid * RPT + k * 64
            pltpu.sync_copy(nd_sp.at[pl.ds(r0, sz)],
                            nd_out.at[t, pl.ds(r0, sz)])
        plsc.subcore_barrier()
        return carry
    lax.fori_loop(0, TPC, _t_step, 0)


def _sc_gat(src_p, dst_p, ae_flat, asrc_f, adst_f, xp_flat):
    mesh = plsc.VectorSubcoreMesh(core_axis_name="c", subcore_axis_name="s")
    kfn = pl.kernel(
        _sc_body,
        out_type=(
            jax.ShapeDtypeStruct((T, N_PAD, ROW), jnp.float32),
            jax.ShapeDtypeStruct((N_PAD, 16), jnp.float32),
        ),
        mesh=mesh,
        scratch_types=[
            pltpu.VMEM((N_PAD * 4,), jnp.float32),
            pltpu.VMEM((N_PAD * 4,), jnp.float32),
            pltpu.VMEM((CHUNK,), jnp.int32),
            pltpu.VMEM((CHUNK,), jnp.int32),
            pltpu.VMEM((CHUNK,), jnp.int32),
            pltpu.VMEM((CHUNK * 4,), jnp.float32),
            pltpu.VMEM((CHUNK, H), jnp.float32),
            pltpu.VMEM((CHUNK, ROW), jnp.float32),
            pltpu.VMEM((CHUNK, 16), jnp.float32),
            pltpu.VMEM((64, ROW), jnp.float32),
            pltpu.VMEM_SHARED((N_PAD, ROW), jnp.float32),
            pltpu.VMEM_SHARED((N_PAD, 16), jnp.float32),
            pltpu.SemaphoreType.DMA,
        ],
    )
    return kfn(src_p, dst_p, ae_flat, asrc_f, adst_f, xp_flat)


def _b_body(nd_ref, hf_ref, xp_ref, als_ref, ald_ref, deg_ref, exp_ref,
            gb_ref, lng_ref, lnb_ref, wih_ref, whh_ref, bih_ref, bhh_ref,
            wh1_ref, bh1_ref, wh2_ref, bh2_ref, out_ref):
    nd = nd_ref[...]
    num = nd[:, :, :H]
    den4 = nd[:, :, H:H + HEADS]
    degs = deg_ref[...]
    deg = jnp.maximum(degs[:, 0], 1.0)
    ael4 = degs[:, 1:1 + HEADS] / deg[:, None]
    al = als_ref[...] + ald_ref[...] + ael4[None]
    al = jnp.where(al >= 0.0, al, 0.2 * al)
    exl = jnp.exp(al)
    expand = exp_ref[...]
    exl64 = jnp.dot(exl.reshape(T * NBLK, HEADS), expand,
                    precision=_HI).reshape(T, NBLK, H)
    den64 = jnp.dot(den4.reshape(T * NBLK, HEADS), expand,
                    precision=_HI).reshape(T, NBLK, H)
    xp = xp_ref[...]
    agg = (num + exl64 * xp) / (den64 + exl64 + 1e-16)
    y = agg + gb_ref[0] + hf_ref[...]
    mu = jnp.mean(y, axis=-1, keepdims=True)
    var = jnp.mean((y - mu) ** 2, axis=-1, keepdims=True)
    y = (y - mu) / jnp.sqrt(var + 1e-5) * lng_ref[0] + lnb_ref[0]

    wih = wih_ref[...]
    whh = whh_ref[...]
    bih = bih_ref[0]
    bhh = bhh_ref[0]
    hst = jnp.zeros((NBLK, H), jnp.float32)
    dn = (((1,), (1,)), ((), ()))
    for t in range(T):
        x_t = y[t]
        gi = lax.dot_general(x_t, wih, dn, precision=_HI) + bih
        gh = lax.dot_general(hst, whh, dn, precision=_HI) + bhh
        r = jax.nn.sigmoid(gi[:, :H] + gh[:, :H])
        z = jax.nn.sigmoid(gi[:, H:2 * H] + gh[:, H:2 * H])
        n = jnp.tanh(gi[:, 2 * H:] + r * gh[:, 2 * H:])
        hst = (1.0 - z) * n + z * hst
    hid = jnp.maximum(jnp.dot(hst, wh1_ref[...], precision=_HI) + bh1_ref[0],
                      0.0)
    out_ref[...] = jnp.dot(hid, wh2_ref[...], precision=_HI) + bh2_ref[0]


def _tc_b(nd, hflat, xproj, als, ald, degsum, expand, gb, lng, lnb,
          wih, whh, bih, bhh, Wh1, bh1, Wh2p, bh2p):
    def full(shape):
        return pl.BlockSpec(shape, lambda i, _s=shape: tuple(0 for _ in _s))
    return pl.pallas_call(
        _b_body,
        grid=(N // NBLK,),
        in_specs=[
            pl.BlockSpec((T, NBLK, ROW), lambda i: (0, i, 0)),
            pl.BlockSpec((T, NBLK, H), lambda i: (0, i, 0)),
            pl.BlockSpec((T, NBLK, H), lambda i: (0, i, 0)),
            pl.BlockSpec((T, NBLK, HEADS), lambda i: (0, i, 0)),
            pl.BlockSpec((T, NBLK, HEADS), lambda i: (0, i, 0)),
            pl.BlockSpec((NBLK, 16), lambda i: (i, 0)),
            full((HEADS, H)),
            full((1, H)),
            full((1, H)),
            full((1, H)),
            full((3 * H, H)),
            full((3 * H, H)),
            full((1, 3 * H)),
            full((1, 3 * H)),
            full((H, H // 2)),
            full((1, H // 2)),
            full((H // 2, TOUT_PAD)),
            full((1, TOUT_PAD)),
        ],
        out_specs=pl.BlockSpec((NBLK, TOUT_PAD), lambda i: (i, 0)),
        out_shape=jax.ShapeDtypeStruct((N, TOUT_PAD), jnp.float32),
    )(nd, hflat, xproj, als, ald, degsum, expand, gb, lng, lnb,
      wih, whh, bih, bhh, Wh1, bh1, Wh2p, bh2p)


def kernel(x_seq, node_attr, edge_index, edge_attr, W1, b1, We, be, Wg,
           att_src, att_dst, Wle, att_edge, gat_bias, ln_g, ln_b, w_ih,
           w_hh, b_ih, b_hh, Wh1, bh1, Wh2, bh2):
    f32 = jnp.float32
    Ve = jnp.einsum("dhc,hc->dh", Wle.reshape(H, HEADS, C), att_edge[0])
    WeVe = We @ Ve
    beVe = (be @ Ve).reshape(1, HEADS)
    eye = jnp.eye(HEADS, dtype=f32)
    Asrc = (att_src[0][:, :, None] * eye[:, None, :]).reshape(H, HEADS)
    Adst = (att_dst[0][:, :, None] * eye[:, None, :]).reshape(H, HEADS)
    expand = jnp.kron(eye, jnp.ones((1, C), f32))
    w1x = W1[:FD]
    w1s = W1[FD:]
    Wh2p = jnp.pad(Wh2, ((0, 0), (0, TOUT_PAD - TOUT)))
    bh2p = jnp.pad(bh2, (0, TOUT_PAD - TOUT)).reshape(1, TOUT_PAD)

    src = edge_index[0]
    dst = edge_index[1]
    npad = E_PAD - E
    src_p = jnp.concatenate([src, jnp.zeros((npad,), jnp.int32)])
    dst_p = jnp.concatenate([dst, jnp.full((npad,), N, jnp.int32)])
    ea_p = jnp.concatenate([edge_attr, jnp.zeros((npad, FE), f32)])

    x_seq3 = x_seq.reshape(T, N, FD)

    hflat, xproj, als, ald = _tc_a1(
        x_seq3, node_attr, w1x, w1s, b1.reshape(1, H), Wg, Asrc, Adst)
    aedge = _tc_a2(ea_p, WeVe, beVe)

    nd, degsum = _sc_gat(
        src_p, dst_p, aedge.reshape(E_PAD * HEADS),
        als.reshape(T, N * HEADS), ald.reshape(T, N * HEADS),
        xproj.reshape(T * N, H))

    pred = _tc_b(
        nd, hflat, xproj, als, ald, degsum, expand,
        gat_bias.reshape(1, H), ln_g.reshape(1, H), ln_b.reshape(1, H),
        w_ih, w_hh, b_ih.reshape(1, 3 * H), b_hh.reshape(1, 3 * H),
        Wh1, bh1.reshape(1, H // 2), Wh2p, bh2p)

    return pred[:, :TOUT].transpose(1, 0).reshape(B, TOUT, N)


# speedup vs baseline: 62.9531x; 62.9531x over previous
import jax
import jax.numpy as jnp
from jax import lax
from jax.experimental import pallas as pl
from jax.experimental.pallas import tpu as pltpu
from jax.experimental.pallas import tpu_sc as plsc

B, T, N = 1, 8, 10000
FD, FS, FE = 8, 16, 4
H, HEADS, TOUT = 64, 4, 6
C = H // HEADS
E = 160000

NCORE, NSUB = 2, 16
CHUNK = 128
NCHUNK = (E + NSUB * CHUNK - 1) // (NSUB * CHUNK)
EPT = NCHUNK * CHUNK
E_PAD = NSUB * EPT
N_PAD = 10112
RPT = N_PAD // NSUB
ROW = 80
TPC = T // NCORE
XPAD = 4
XPA = H + HEADS + XPAD
NBLK = 1000
NBLK_B = 400
EBLK = 2048
TOUT_PAD = 8

_HI = jax.lax.Precision.HIGHEST


def _a1_body(x_ref, na_ref, w1x_ref, w1s_ref, b1_ref, wg_ref, asrc_ref,
             adst_ref, hf_ref, xp_ref, als_ref, ald_ref, xpa_ref, ald8_ref):
    x = x_ref[0]
    na = na_ref[...]
    h = jnp.dot(x, w1x_ref[...], precision=_HI)
    h = h + jnp.dot(na, w1s_ref[...], precision=_HI) + b1_ref[0]
    h = jnp.maximum(h, 0.0)
    xp = jnp.dot(h, wg_ref[...], precision=_HI)
    hf_ref[0] = h
    xp_ref[0] = xp
    als = jnp.dot(xp, asrc_ref[...], precision=_HI)
    ald = jnp.dot(xp, adst_ref[...], precision=_HI)
    als_ref[0] = als
    ald_ref[0] = ald
    z4 = jnp.zeros((NBLK, XPAD), jnp.float32)
    xpa_ref[0] = jnp.concatenate([xp, als, z4], axis=1)
    ald8_ref[0] = jnp.concatenate([ald, z4], axis=1)


def _tc_a1(x_seq3, node_attr, w1x, w1s, b1r, Wg, Asrc, Adst):
    grid = (T, N // NBLK)
    out_shape = (
        jax.ShapeDtypeStruct((T, N, H), jnp.float32),
        jax.ShapeDtypeStruct((T, N, H), jnp.float32),
        jax.ShapeDtypeStruct((T, N, HEADS), jnp.float32),
        jax.ShapeDtypeStruct((T, N, HEADS), jnp.float32),
        jax.ShapeDtypeStruct((T, N, XPA), jnp.float32),
        jax.ShapeDtypeStruct((T, N, 2 * HEADS), jnp.float32),
    )
    return pl.pallas_call(
        _a1_body,
        grid=grid,
        in_specs=[
            pl.BlockSpec((1, NBLK, FD), lambda t, i: (t, i, 0)),
            pl.BlockSpec((NBLK, FS), lambda t, i: (i, 0)),
            pl.BlockSpec((FD, H), lambda t, i: (0, 0)),
            pl.BlockSpec((FS, H), lambda t, i: (0, 0)),
            pl.BlockSpec((1, H), lambda t, i: (0, 0)),
            pl.BlockSpec((H, H), lambda t, i: (0, 0)),
            pl.BlockSpec((H, HEADS), lambda t, i: (0, 0)),
            pl.BlockSpec((H, HEADS), lambda t, i: (0, 0)),
        ],
        out_specs=(
            pl.BlockSpec((1, NBLK, H), lambda t, i: (t, i, 0)),
            pl.BlockSpec((1, NBLK, H), lambda t, i: (t, i, 0)),
            pl.BlockSpec((1, NBLK, HEADS), lambda t, i: (t, i, 0)),
            pl.BlockSpec((1, NBLK, HEADS), lambda t, i: (t, i, 0)),
            pl.BlockSpec((1, NBLK, XPA), lambda t, i: (t, i, 0)),
            pl.BlockSpec((1, NBLK, 2 * HEADS), lambda t, i: (t, i, 0)),
        ),
        out_shape=out_shape,
    )(x_seq3, node_attr, w1x, w1s, b1r, Wg, Asrc, Adst)


def _a2_body(ea_ref, weve_ref, beve_ref, out_ref):
    out_ref[...] = (
        jnp.dot(ea_ref[...], weve_ref[...], precision=_HI) + beve_ref[0]
    )


def _tc_a2(ea_pad, WeVe, beVe_r):
    return pl.pallas_call(
        _a2_body,
        grid=(E_PAD // EBLK,),
        in_specs=[
            pl.BlockSpec((EBLK, FE), lambda i: (i, 0)),
            pl.BlockSpec((FE, HEADS), lambda i: (0, 0)),
            pl.BlockSpec((1, HEADS), lambda i: (0, 0)),
        ],
        out_specs=pl.BlockSpec((EBLK, HEADS), lambda i: (i, 0)),
        out_shape=jax.ShapeDtypeStruct((E_PAD, HEADS), jnp.float32),
    )(ea_pad, WeVe, beVe_r)


def _sc_body(src_h, dst_h, ae_h, xpa_h, ald_h,
             nd_out, deg_out,
             src_c, dst_c, idx_c, idx2_c, ae_c, xg, adg, msg,
             zrow, nd_sp, sem, sem2):
    cid = lax.axis_index("c")
    sid = lax.axis_index("s")
    ebase = sid * EPT
    zeros16 = jnp.zeros((16,), jnp.float32)

    def _zrow_init(i, carry):
        r = i // 5
        c = (i % 5) * 16
        zrow[r, pl.ds(c, 16)] = zeros16
        return carry
    lax.fori_loop(0, 64 * 5, _zrow_init, 0)

    def _msg_init(k, carry):
        for c in range(ROW // 16):
            msg[k, pl.ds(c * 16, 16)] = zeros16
        return carry
    lax.fori_loop(0, CHUNK, _msg_init, 0)

    @pl.when(cid == 0)
    def _phase0():
        for k in range(10):
            sz = 64 if k < 9 else RPT - 9 * 64
            pltpu.sync_copy(zrow.at[pl.ds(0, sz)],
                            nd_sp.at[pl.ds(sid * RPT + k * 64, sz)])
        plsc.subcore_barrier()

        iota16 = lax.iota(jnp.int32, 16)
        lane_eq = [iota16 == h for h in range(5)]
        base_row = jnp.where(lane_eq[0], 1.0, 0.0)

        def _p0_chunk(j, carry):
            base = ebase + j * CHUNK
            pltpu.sync_copy(dst_h.at[pl.ds(base, CHUNK)], dst_c)
            pltpu.sync_copy(ae_h.at[pl.ds(base * 4, CHUNK * 4)], ae_c)

            def _p0_group(g, c2):
                k16 = iota16 + g * 16
                aevecs = [plsc.load_gather(ae_c, [k16 * 4 + h])
                          for h in range(HEADS)]
                for i in range(16):
                    k = g * 16 + i
                    v = base_row
                    for h in range(HEADS):
                        v = jnp.where(lane_eq[1 + h],
                                      jnp.full((16,), aevecs[h][i]), v)
                    msg[k, pl.ds(0, 16)] = v
                return c2
            lax.fori_loop(0, 8, _p0_group, 0)
            pltpu.sync_copy(msg, nd_sp.at[dst_c], add=True)
            return carry
        lax.fori_loop(0, NCHUNK, _p0_chunk, 0)
        plsc.subcore_barrier()
        for k in range(10):
            sz = 64 if k < 9 else RPT - 9 * 64
            r0 = sid * RPT + k * 64
            pltpu.sync_copy(nd_sp.at[pl.ds(r0, sz)],
                            deg_out.at[pl.ds(r0, sz)])

    def _t_step(jt, carry):
        t = cid * TPC + jt
        for k in range(10):
            sz = 64 if k < 9 else RPT - 9 * 64
            pltpu.sync_copy(zrow.at[pl.ds(0, sz)],
                            nd_sp.at[pl.ds(sid * RPT + k * 64, sz)])
        plsc.subcore_barrier()

        tN = t * N

        def _chunk(j, carry2):
            base = ebase + j * CHUNK
            pltpu.sync_copy(src_h.at[pl.ds(base, CHUNK)], src_c)
            pltpu.sync_copy(dst_h.at[pl.ds(base, CHUNK)], dst_c)
            pltpu.sync_copy(ae_h.at[pl.ds(base * 4, CHUNK * 4)], ae_c)
            for g in range(8):
                idx_c[pl.ds(g * 16, 16)] = src_c[pl.ds(g * 16, 16)] + tN
                idx2_c[pl.ds(g * 16, 16)] = dst_c[pl.ds(g * 16, 16)] + tN
            cp1 = pltpu.async_copy(xpa_h.at[idx_c], xg, sem)
            cp2 = pltpu.async_copy(ald_h.at[idx2_c], adg, sem2)
            cp1.wait()
            cp2.wait()
            iota16 = lax.iota(jnp.int32, 16)
            lane_eq = [iota16 == h for h in range(HEADS)]
            zv = jnp.zeros((16,), jnp.float32)

            def _group(g, c3):
                k16 = iota16 + g * 16
                exvecs = []
                for h in range(HEADS):
                    hv = jnp.full((16,), h, jnp.int32)
                    a = (plsc.load_gather(xg, [k16, hv + H])
                         + plsc.load_gather(adg, [k16, hv])
                         + plsc.load_gather(ae_c, [k16 * 4 + h]))
                    a = jnp.where(a >= 0.0, a, 0.2 * a)
                    exvecs.append(jnp.exp(a))
                for i in range(16):
                    k = g * 16 + i
                    exv = zv
                    for h in range(HEADS):
                        sf = jnp.full((16,), exvecs[h][i])
                        msg[k, pl.ds(h * C, 16)] = xg[k, pl.ds(h * C, 16)] * sf
                        exv = jnp.where(lane_eq[h], sf, exv)
                    msg[k, pl.ds(H, 16)] = exv
                return c3
            lax.fori_loop(0, 8, _group, 0)
            pltpu.sync_copy(msg, nd_sp.at[dst_c], add=True)
            return carry2
        lax.fori_loop(0, NCHUNK, _chunk, 0)
        plsc.subcore_barrier()
        for k in range(10):
            sz = 64 if k < 9 else RPT - 9 * 64
            r0 = sid * RPT + k * 64
            pltpu.sync_copy(nd_sp.at[pl.ds(r0, sz)],
                            nd_out.at[t, pl.ds(r0, sz)])
        plsc.subcore_barrier()
        return carry
    lax.fori_loop(0, TPC, _t_step, 0)


def _sc_gat(src_p, dst_p, ae_flat, xpa_f, ald_f):
    mesh = plsc.VectorSubcoreMesh(core_axis_name="c", subcore_axis_name="s",
                                  num_cores=NCORE, num_subcores=NSUB)
    kfn = pl.kernel(
        _sc_body,
        out_type=(
            jax.ShapeDtypeStruct((T, N_PAD, ROW), jnp.float32),
            jax.ShapeDtypeStruct((N_PAD, ROW), jnp.float32),
        ),
        mesh=mesh,
        scratch_types=[
            pltpu.VMEM((CHUNK,), jnp.int32),
            pltpu.VMEM((CHUNK,), jnp.int32),
            pltpu.VMEM((CHUNK,), jnp.int32),
            pltpu.VMEM((CHUNK,), jnp.int32),
            pltpu.VMEM((CHUNK * 4,), jnp.float32),
            pltpu.VMEM((CHUNK, XPA), jnp.float32),
            pltpu.VMEM((CHUNK, 2 * HEADS), jnp.float32),
            pltpu.VMEM((CHUNK, ROW), jnp.float32),
            pltpu.VMEM((64, ROW), jnp.float32),
            pltpu.VMEM_SHARED((N_PAD, ROW), jnp.float32),
            pltpu.SemaphoreType.DMA,
            pltpu.SemaphoreType.DMA,
        ],
        compiler_params=pltpu.CompilerParams(needs_layout_passes=False,
                                             use_tc_tiling_on_sc=False),
    )
    return kfn(src_p, dst_p, ae_flat, xpa_f, ald_f)


def _b_body(nd_ref, hf_ref, xp_ref, als_ref, ald_ref, deg_ref, exp_ref,
            gb_ref, lng_ref, lnb_ref, wih_ref, whh_ref, bih_ref, bhh_ref,
            wh1_ref, bh1_ref, wh2_ref, bh2_ref, out_ref):
    nd = nd_ref[...]
    num = nd[:, :, :H]
    den4 = nd[:, :, H:H + HEADS]
    degs = deg_ref[...]
    deg = jnp.maximum(degs[:, 0], 1.0)
    ael4 = degs[:, 1:1 + HEADS] / deg[:, None]
    al = als_ref[...] + ald_ref[...] + ael4[None]
    al = jnp.where(al >= 0.0, al, 0.2 * al)
    exl = jnp.exp(al)
    expand = exp_ref[...]
    exl64 = jnp.dot(exl.reshape(T * NBLK_B, HEADS), expand,
                    precision=_HI).reshape(T, NBLK_B, H)
    den64 = jnp.dot(den4.reshape(T * NBLK_B, HEADS), expand,
                    precision=_HI).reshape(T, NBLK_B, H)
    xp = xp_ref[...]
    agg = (num + exl64 * xp) / (den64 + exl64 + 1e-16)
    y = agg + gb_ref[0] + hf_ref[...]
    mu = jnp.mean(y, axis=-1, keepdims=True)
    var = jnp.mean((y - mu) ** 2, axis=-1, keepdims=True)
    y = (y - mu) / jnp.sqrt(var + 1e-5) * lng_ref[0] + lnb_ref[0]

    wih = wih_ref[...]
    whh = whh_ref[...]
    bih = bih_ref[0]
    bhh = bhh_ref[0]
    hst = jnp.zeros((NBLK_B, H), jnp.float32)
    dn = (((1,), (1,)), ((), ()))
    for t in range(T):
        x_t = y[t]
        gi = lax.dot_general(x_t, wih, dn, precision=_HI) + bih
        gh = lax.dot_general(hst, whh, dn, precision=_HI) + bhh
        r = jax.nn.sigmoid(gi[:, :H] + gh[:, :H])
        z = jax.nn.sigmoid(gi[:, H:2 * H] + gh[:, H:2 * H])
        n = jnp.tanh(gi[:, 2 * H:] + r * gh[:, 2 * H:])
        hst = (1.0 - z) * n + z * hst
    hid = jnp.maximum(jnp.dot(hst, wh1_ref[...], precision=_HI) + bh1_ref[0],
                      0.0)
    out_ref[...] = jnp.dot(hid, wh2_ref[...], precision=_HI) + bh2_ref[0]


def _tc_b(nd, hflat, xproj, als, ald, degsum, expand, gb, lng, lnb,
          wih, whh, bih, bhh, Wh1, bh1, Wh2p, bh2p):
    def full(shape):
        return pl.BlockSpec(shape, lambda i, _s=shape: tuple(0 for _ in _s))
    return pl.pallas_call(
        _b_body,
        grid=(N // NBLK_B,),
        in_specs=[
            pl.BlockSpec((T, NBLK_B, ROW), lambda i: (0, i, 0)),
            pl.BlockSpec((T, NBLK_B, H), lambda i: (0, i, 0)),
            pl.BlockSpec((T, NBLK_B, H), lambda i: (0, i, 0)),
            pl.BlockSpec((T, NBLK_B, HEADS), lambda i: (0, i, 0)),
            pl.BlockSpec((T, NBLK_B, HEADS), lambda i: (0, i, 0)),
            pl.BlockSpec((NBLK_B, ROW), lambda i: (i, 0)),
            full((HEADS, H)),
            full((1, H)),
            full((1, H)),
            full((1, H)),
            full((3 * H, H)),
            full((3 * H, H)),
            full((1, 3 * H)),
            full((1, 3 * H)),
            full((H, H // 2)),
            full((1, H // 2)),
            full((H // 2, TOUT_PAD)),
            full((1, TOUT_PAD)),
        ],
        out_specs=pl.BlockSpec((NBLK_B, TOUT_PAD), lambda i: (i, 0)),
        out_shape=jax.ShapeDtypeStruct((N, TOUT_PAD), jnp.float32),
    )(nd, hflat, xproj, als, ald, degsum, expand, gb, lng, lnb,
      wih, whh, bih, bhh, Wh1, bh1, Wh2p, bh2p)


def kernel(x_seq, node_attr, edge_index, edge_attr, W1, b1, We, be, Wg,
           att_src, att_dst, Wle, att_edge, gat_bias, ln_g, ln_b, w_ih,
           w_hh, b_ih, b_hh, Wh1, bh1, Wh2, bh2):
    f32 = jnp.float32
    Ve = jnp.einsum("dhc,hc->dh", Wle.reshape(H, HEADS, C), att_edge[0])
    WeVe = We @ Ve
    beVe = (be @ Ve).reshape(1, HEADS)
    eye = jnp.eye(HEADS, dtype=f32)
    Asrc = (att_src[0][:, :, None] * eye[:, None, :]).reshape(H, HEADS)
    Adst = (att_dst[0][:, :, None] * eye[:, None, :]).reshape(H, HEADS)
    expand = jnp.kron(eye, jnp.ones((1, C), f32))
    w1x = W1[:FD]
    w1s = W1[FD:]
    Wh2p = jnp.pad(Wh2, ((0, 0), (0, TOUT_PAD - TOUT)))
    bh2p = jnp.pad(bh2, (0, TOUT_PAD - TOUT)).reshape(1, TOUT_PAD)

    src = edge_index[0]
    dst = edge_index[1]
    npad = E_PAD - E
    src_p = jnp.concatenate([src, jnp.zeros((npad,), jnp.int32)])
    dst_p = jnp.concatenate([dst, jnp.full((npad,), N, jnp.int32)])
    ea_p = jnp.concatenate([edge_attr, jnp.zeros((npad, FE), f32)])

    x_seq3 = x_seq.reshape(T, N, FD)

    hflat, xproj, als, ald, xpa, ald8 = _tc_a1(
        x_seq3, node_attr, w1x, w1s, b1.reshape(1, H), Wg, Asrc, Adst)
    aedge = _tc_a2(ea_p, WeVe, beVe)

    ald8_f = jnp.concatenate(
        [ald8.reshape(T * N, 2 * HEADS),
         jnp.zeros((128, 2 * HEADS), f32)])

    nd, degsum = _sc_gat(
        src_p, dst_p, aedge.reshape(E_PAD * HEADS),
        xpa.reshape(T * N, XPA), ald8_f)

    pred = _tc_b(
        nd, hflat, xproj, als, ald, degsum, expand,
        gat_bias.reshape(1, H), ln_g.reshape(1, H), ln_b.reshape(1, H),
        w_ih, w_hh, b_ih.reshape(1, 3 * H), b_hh.reshape(1, 3 * H),
        Wh1, bh1.reshape(1, H // 2), Wh2p, bh2p)

    return pred[:, :TOUT].transpose(1, 0).reshape(B, TOUT, N)


# speedup vs baseline: 83.4659x; 1.3258x over previous
import jax
import jax.numpy as jnp
from jax import lax
from jax.experimental import pallas as pl
from jax.experimental.pallas import tpu as pltpu
from jax.experimental.pallas import tpu_sc as plsc

B, T, N = 1, 8, 10000
FD, FS, FE = 8, 16, 4
H, HEADS, TOUT = 64, 4, 6
C = H // HEADS
E = 160000

NCORE, NSUB = 2, 16
CHUNK = 128
NCHUNK = 80
EPT = NCHUNK * CHUNK
E_PAD = NSUB * EPT
N_PAD = 10112
RPT = N_PAD // NSUB
ROW = 80
TPC = T // NCORE
XPAD = 4
XPA = H + HEADS + XPAD
NBLK = 1000
NBLK_B = 400
EBLK = 2048
TOUT_PAD = 8

_HI = jax.lax.Precision.HIGHEST


def _a1_body(x_ref, na_ref, w1x_ref, w1s_ref, b1_ref, wg_ref, asrc_ref,
             adst_ref, hf_ref, xp_ref, als_ref, ald_ref, xpa_ref, ald8_ref):
    x = x_ref[0]
    na = na_ref[...]
    h = jnp.dot(x, w1x_ref[...], precision=_HI)
    h = h + jnp.dot(na, w1s_ref[...], precision=_HI) + b1_ref[0]
    h = jnp.maximum(h, 0.0)
    xp = jnp.dot(h, wg_ref[...], precision=_HI)
    hf_ref[0] = h
    xp_ref[0] = xp
    als = jnp.dot(xp, asrc_ref[...], precision=_HI)
    ald = jnp.dot(xp, adst_ref[...], precision=_HI)
    als_ref[0] = als
    ald_ref[0] = ald
    z4 = jnp.zeros((NBLK, XPAD), jnp.float32)
    xpa_ref[0] = jnp.concatenate([xp, als, z4], axis=1)
    ald8_ref[0] = jnp.concatenate([ald, z4], axis=1)


def _tc_a1(x_seq3, node_attr, w1x, w1s, b1r, Wg, Asrc, Adst):
    grid = (T, N // NBLK)
    out_shape = (
        jax.ShapeDtypeStruct((T, N, H), jnp.float32),
        jax.ShapeDtypeStruct((T, N, H), jnp.float32),
        jax.ShapeDtypeStruct((T, N, HEADS), jnp.float32),
        jax.ShapeDtypeStruct((T, N, HEADS), jnp.float32),
        jax.ShapeDtypeStruct((T, N, XPA), jnp.float32),
        jax.ShapeDtypeStruct((T, N, 2 * HEADS), jnp.float32),
    )
    return pl.pallas_call(
        _a1_body,
        grid=grid,
        in_specs=[
            pl.BlockSpec((1, NBLK, FD), lambda t, i: (t, i, 0)),
            pl.BlockSpec((NBLK, FS), lambda t, i: (i, 0)),
            pl.BlockSpec((FD, H), lambda t, i: (0, 0)),
            pl.BlockSpec((FS, H), lambda t, i: (0, 0)),
            pl.BlockSpec((1, H), lambda t, i: (0, 0)),
            pl.BlockSpec((H, H), lambda t, i: (0, 0)),
            pl.BlockSpec((H, HEADS), lambda t, i: (0, 0)),
            pl.BlockSpec((H, HEADS), lambda t, i: (0, 0)),
        ],
        out_specs=(
            pl.BlockSpec((1, NBLK, H), lambda t, i: (t, i, 0)),
            pl.BlockSpec((1, NBLK, H), lambda t, i: (t, i, 0)),
            pl.BlockSpec((1, NBLK, HEADS), lambda t, i: (t, i, 0)),
            pl.BlockSpec((1, NBLK, HEADS), lambda t, i: (t, i, 0)),
            pl.BlockSpec((1, NBLK, XPA), lambda t, i: (t, i, 0)),
            pl.BlockSpec((1, NBLK, 2 * HEADS), lambda t, i: (t, i, 0)),
        ),
        out_shape=out_shape,
    )(x_seq3, node_attr, w1x, w1s, b1r, Wg, Asrc, Adst)


def _a2_body(ea_ref, weve_ref, beve_ref, out_ref):
    out_ref[...] = (
        jnp.dot(ea_ref[...], weve_ref[...], precision=_HI) + beve_ref[0]
    )


def _tc_a2(ea_pad, WeVe, beVe_r):
    return pl.pallas_call(
        _a2_body,
        grid=(E_PAD // EBLK,),
        in_specs=[
            pl.BlockSpec((EBLK, FE), lambda i: (i, 0)),
            pl.BlockSpec((FE, HEADS), lambda i: (0, 0)),
            pl.BlockSpec((1, HEADS), lambda i: (0, 0)),
        ],
        out_specs=pl.BlockSpec((EBLK, HEADS), lambda i: (i, 0)),
        out_shape=jax.ShapeDtypeStruct((E_PAD, HEADS), jnp.float32),
    )(ea_pad, WeVe, beVe_r)


def _sc_body(src_h, dst_h, ae_h, xpa_h, ald_h,
             nd_out, deg_out,
             src_t, dst_t, idx_t, idx2_t,
             ae0, ae1, xg0, xg1, adg0, adg1, msg,
             zrow, nd_sp, se0, se1, sx0, sx1, sa0, sa1):
    cid = lax.axis_index("c")
    sid = lax.axis_index("s")
    rbase = sid * NCHUNK
    zeros16 = jnp.zeros((16,), jnp.float32)
    iota16 = lax.iota(jnp.int32, 16)
    bufs = ((ae0, xg0, adg0, se0, sx0, sa0),
            (ae1, xg1, adg1, se1, sx1, sa1))

    def _zrow_init(i, carry):
        r = i // 5
        c = (i % 5) * 16
        zrow[r, pl.ds(c, 16)] = zeros16
        return carry
    lax.fori_loop(0, 64 * 5, _zrow_init, 0)

    def _msg_init(k, carry):
        for c in range(ROW // 16):
            msg[k, pl.ds(c * 16, 16)] = zeros16
        return carry
    lax.fori_loop(0, CHUNK, _msg_init, 0)

    pltpu.sync_copy(src_h.at[pl.ds(rbase, NCHUNK)], src_t)
    pltpu.sync_copy(dst_h.at[pl.ds(rbase, NCHUNK)], dst_t)

    def _zero_nd_slice():
        for k in range(10):
            sz = 64 if k < 9 else RPT - 9 * 64
            pltpu.sync_copy(zrow.at[pl.ds(0, sz)],
                            nd_sp.at[pl.ds(sid * RPT + k * 64, sz)])

    @pl.when(cid == 0)
    def _phase0():
        _zero_nd_slice()
        plsc.subcore_barrier()

        lane_eq5 = [iota16 == h for h in range(5)]
        base_row = jnp.where(lane_eq5[0], 1.0, 0.0)

        def _p0_chunk(j, carry):
            pltpu.sync_copy(ae_h.at[rbase + j], ae0)

            def _p0_group(g, c2):
                k16 = iota16 + g * 16
                aevecs = [plsc.load_gather(ae0, [k16 * 4 + h])
                          for h in range(HEADS)]
                for i in range(16):
                    k = g * 16 + i
                    v = base_row
                    for h in range(HEADS):
                        v = jnp.where(lane_eq5[1 + h],
                                      jnp.full((16,), aevecs[h][i]), v)
                    msg[k, pl.ds(0, 16)] = v
                return c2
            lax.fori_loop(0, 8, _p0_group, 0)
            pltpu.sync_copy(msg, nd_sp.at[dst_t.at[j]], add=True)
            return carry
        lax.fori_loop(0, NCHUNK, _p0_chunk, 0)
        plsc.subcore_barrier()
        for k in range(10):
            sz = 64 if k < 9 else RPT - 9 * 64
            r0 = sid * RPT + k * 64
            pltpu.sync_copy(nd_sp.at[pl.ds(r0, sz)],
                            deg_out.at[pl.ds(r0, sz)])

    def _start(j, b):
        ae_b, xg_b, adg_b, se_b, sx_b, sa_b = bufs[b]
        pltpu.async_copy(ae_h.at[rbase + j], ae_b, se_b)
        pltpu.async_copy(xpa_h.at[idx_t.at[j]], xg_b, sx_b)
        pltpu.async_copy(ald_h.at[idx2_t.at[j]], adg_b, sa_b)

    def _wait(j, b):
        ae_b, xg_b, adg_b, se_b, sx_b, sa_b = bufs[b]
        pltpu.make_async_copy(ae_h.at[rbase + j], ae_b, se_b).wait()
        pltpu.make_async_copy(xpa_h.at[idx_t.at[j]], xg_b, sx_b).wait()
        pltpu.make_async_copy(ald_h.at[idx2_t.at[j]], adg_b, sa_b).wait()

    lane_eq = [iota16 == h for h in range(HEADS)]
    zv = jnp.zeros((16,), jnp.float32)

    def _compute(j, b):
        ae_b, xg_b, adg_b = bufs[b][0], bufs[b][1], bufs[b][2]

        def _group(g, c3):
            k16 = iota16 + g * 16
            exvecs = []
            for h in range(HEADS):
                hv = jnp.full((16,), h, jnp.int32)
                a = (plsc.load_gather(xg_b, [k16, hv + H])
                     + plsc.load_gather(adg_b, [k16, hv])
                     + plsc.load_gather(ae_b, [k16 * 4 + h]))
                a = jnp.where(a >= 0.0, a, 0.2 * a)
                exvecs.append(jnp.exp(a))
            for i in range(16):
                k = g * 16 + i
                exv = zv
                for h in range(HEADS):
                    sf = jnp.full((16,), exvecs[h][i])
                    msg[k, pl.ds(h * C, 16)] = xg_b[k, pl.ds(h * C, 16)] * sf
                    exv = jnp.where(lane_eq[h], sf, exv)
                msg[k, pl.ds(H, 16)] = exv
            return c3
        lax.fori_loop(0, 8, _group, 0)
        pltpu.sync_copy(msg, nd_sp.at[dst_t.at[j]], add=True)

    def _t_step(jt, carry):
        t = cid * TPC + jt
        tN = t * N
        _zero_nd_slice()

        def _idx(i, c):
            r = i // 8
            c16 = (i % 8) * 16
            idx_t[r, pl.ds(c16, 16)] = src_t[r, pl.ds(c16, 16)] + tN
            idx2_t[r, pl.ds(c16, 16)] = dst_t[r, pl.ds(c16, 16)] + tN
            return c
        lax.fori_loop(0, NCHUNK * 8, _idx, 0)
        plsc.subcore_barrier()

        _start(0, 0)

        def _pair(g, carry2):
            for b in range(2):
                j = 2 * g + b

                @pl.when(j + 1 < NCHUNK)
                def _pre():
                    _start(j + 1, 1 - b)
                _wait(j, b)
                _compute(j, b)
            return carry2
        lax.fori_loop(0, NCHUNK // 2, _pair, 0)
        plsc.subcore_barrier()
        for k in range(10):
            sz = 64 if k < 9 else RPT - 9 * 64
            r0 = sid * RPT + k * 64
            pltpu.sync_copy(nd_sp.at[pl.ds(r0, sz)],
                            nd_out.at[t, pl.ds(r0, sz)])
        plsc.subcore_barrier()
        return carry
    lax.fori_loop(0, TPC, _t_step, 0)


def _sc_gat(src2, dst2, ae2, xpa_f, ald_f):
    mesh = plsc.VectorSubcoreMesh(core_axis_name="c", subcore_axis_name="s",
                                  num_cores=NCORE, num_subcores=NSUB)
    kfn = pl.kernel(
        _sc_body,
        out_type=(
            jax.ShapeDtypeStruct((T, N_PAD, ROW), jnp.float32),
            jax.ShapeDtypeStruct((N_PAD, ROW), jnp.float32),
        ),
        mesh=mesh,
        scratch_types=[
            pltpu.VMEM((NCHUNK, CHUNK), jnp.int32),
            pltpu.VMEM((NCHUNK, CHUNK), jnp.int32),
            pltpu.VMEM((NCHUNK, CHUNK), jnp.int32),
            pltpu.VMEM((NCHUNK, CHUNK), jnp.int32),
            pltpu.VMEM((CHUNK * 4,), jnp.float32),
            pltpu.VMEM((CHUNK * 4,), jnp.float32),
            pltpu.VMEM((CHUNK, XPA), jnp.float32),
            pltpu.VMEM((CHUNK, XPA), jnp.float32),
            pltpu.VMEM((CHUNK, 2 * HEADS), jnp.float32),
            pltpu.VMEM((CHUNK, 2 * HEADS), jnp.float32),
            pltpu.VMEM((CHUNK, ROW), jnp.float32),
            pltpu.VMEM((64, ROW), jnp.float32),
            pltpu.VMEM_SHARED((N_PAD, ROW), jnp.float32),
            pltpu.SemaphoreType.DMA,
            pltpu.SemaphoreType.DMA,
            pltpu.SemaphoreType.DMA,
            pltpu.SemaphoreType.DMA,
            pltpu.SemaphoreType.DMA,
            pltpu.SemaphoreType.DMA,
        ],
        compiler_params=pltpu.CompilerParams(needs_layout_passes=False,
                                             use_tc_tiling_on_sc=False),
    )
    return kfn(src2, dst2, ae2, xpa_f, ald_f)


def _b_body(nd_ref, hf_ref, xp_ref, als_ref, ald_ref, deg_ref, exp_ref,
            gb_ref, lng_ref, lnb_ref, wih_ref, whh_ref, bih_ref, bhh_ref,
            wh1_ref, bh1_ref, wh2_ref, bh2_ref, out_ref):
    nd = nd_ref[...]
    num = nd[:, :, :H]
    den4 = nd[:, :, H:H + HEADS]
    degs = deg_ref[...]
    deg = jnp.maximum(degs[:, 0], 1.0)
    ael4 = degs[:, 1:1 + HEADS] / deg[:, None]
    al = als_ref[...] + ald_ref[...] + ael4[None]
    al = jnp.where(al >= 0.0, al, 0.2 * al)
    exl = jnp.exp(al)
    expand = exp_ref[...]
    exl64 = jnp.dot(exl.reshape(T * NBLK_B, HEADS), expand,
                    precision=_HI).reshape(T, NBLK_B, H)
    den64 = jnp.dot(den4.reshape(T * NBLK_B, HEADS), expand,
                    precision=_HI).reshape(T, NBLK_B, H)
    xp = xp_ref[...]
    agg = (num + exl64 * xp) / (den64 + exl64 + 1e-16)
    y = agg + gb_ref[0] + hf_ref[...]
    mu = jnp.mean(y, axis=-1, keepdims=True)
    var = jnp.mean((y - mu) ** 2, axis=-1, keepdims=True)
    y = (y - mu) / jnp.sqrt(var + 1e-5) * lng_ref[0] + lnb_ref[0]

    wih = wih_ref[...]
    whh = whh_ref[...]
    bih = bih_ref[0]
    bhh = bhh_ref[0]
    hst = jnp.zeros((NBLK_B, H), jnp.float32)
    dn = (((1,), (1,)), ((), ()))
    for t in range(T):
        x_t = y[t]
        gi = lax.dot_general(x_t, wih, dn, precision=_HI) + bih
        gh = lax.dot_general(hst, whh, dn, precision=_HI) + bhh
        r = jax.nn.sigmoid(gi[:, :H] + gh[:, :H])
        z = jax.nn.sigmoid(gi[:, H:2 * H] + gh[:, H:2 * H])
        n = jnp.tanh(gi[:, 2 * H:] + r * gh[:, 2 * H:])
        hst = (1.0 - z) * n + z * hst
    hid = jnp.maximum(jnp.dot(hst, wh1_ref[...], precision=_HI) + bh1_ref[0],
                      0.0)
    out_ref[...] = jnp.dot(hid, wh2_ref[...], precision=_HI) + bh2_ref[0]


def _tc_b(nd, hflat, xproj, als, ald, degsum, expand, gb, lng, lnb,
          wih, whh, bih, bhh, Wh1, bh1, Wh2p, bh2p):
    def full(shape):
        return pl.BlockSpec(shape, lambda i, _s=shape: tuple(0 for _ in _s))
    return pl.pallas_call(
        _b_body,
        grid=(N // NBLK_B,),
        in_specs=[
            pl.BlockSpec((T, NBLK_B, ROW), lambda i: (0, i, 0)),
            pl.BlockSpec((T, NBLK_B, H), lambda i: (0, i, 0)),
            pl.BlockSpec((T, NBLK_B, H), lambda i: (0, i, 0)),
            pl.BlockSpec((T, NBLK_B, HEADS), lambda i: (0, i, 0)),
            pl.BlockSpec((T, NBLK_B, HEADS), lambda i: (0, i, 0)),
            pl.BlockSpec((NBLK_B, ROW), lambda i: (i, 0)),
            full((HEADS, H)),
            full((1, H)),
            full((1, H)),
            full((1, H)),
            full((3 * H, H)),
            full((3 * H, H)),
            full((1, 3 * H)),
            full((1, 3 * H)),
            full((H, H // 2)),
            full((1, H // 2)),
            full((H // 2, TOUT_PAD)),
            full((1, TOUT_PAD)),
        ],
        out_specs=pl.BlockSpec((NBLK_B, TOUT_PAD), lambda i: (i, 0)),
        out_shape=jax.ShapeDtypeStruct((N, TOUT_PAD), jnp.float32),
    )(nd, hflat, xproj, als, ald, degsum, expand, gb, lng, lnb,
      wih, whh, bih, bhh, Wh1, bh1, Wh2p, bh2p)


def kernel(x_seq, node_attr, edge_index, edge_attr, W1, b1, We, be, Wg,
           att_src, att_dst, Wle, att_edge, gat_bias, ln_g, ln_b, w_ih,
           w_hh, b_ih, b_hh, Wh1, bh1, Wh2, bh2):
    f32 = jnp.float32
    Ve = jnp.einsum("dhc,hc->dh", Wle.reshape(H, HEADS, C), att_edge[0])
    WeVe = We @ Ve
    beVe = (be @ Ve).reshape(1, HEADS)
    eye = jnp.eye(HEADS, dtype=f32)
    Asrc = (att_src[0][:, :, None] * eye[:, None, :]).reshape(H, HEADS)
    Adst = (att_dst[0][:, :, None] * eye[:, None, :]).reshape(H, HEADS)
    expand = jnp.kron(eye, jnp.ones((1, C), f32))
    w1x = W1[:FD]
    w1s = W1[FD:]
    Wh2p = jnp.pad(Wh2, ((0, 0), (0, TOUT_PAD - TOUT)))
    bh2p = jnp.pad(bh2, (0, TOUT_PAD - TOUT)).reshape(1, TOUT_PAD)

    src = edge_index[0]
    dst = edge_index[1]
    npad = E_PAD - E
    src_p = jnp.concatenate([src, jnp.zeros((npad,), jnp.int32)])
    dst_p = jnp.concatenate([dst, jnp.full((npad,), N, jnp.int32)])
    ea_p = jnp.concatenate([edge_attr, jnp.zeros((npad, FE), f32)])

    x_seq3 = x_seq.reshape(T, N, FD)

    hflat, xproj, als, ald, xpa, ald8 = _tc_a1(
        x_seq3, node_attr, w1x, w1s, b1.reshape(1, H), Wg, Asrc, Adst)
    aedge = _tc_a2(ea_p, WeVe, beVe)

    ald8_f = jnp.concatenate(
        [ald8.reshape(T * N, 2 * HEADS),
         jnp.zeros((128, 2 * HEADS), f32)])

    nd, degsum = _sc_gat(
        src_p.reshape(-1, CHUNK), dst_p.reshape(-1, CHUNK),
        aedge.reshape(-1, CHUNK * HEADS),
        xpa.reshape(T * N, XPA), ald8_f)

    pred = _tc_b(
        nd, hflat, xproj, als, ald, degsum, expand,
        gat_bias.reshape(1, H), ln_g.reshape(1, H), ln_b.reshape(1, H),
        w_ih, w_hh, b_ih.reshape(1, 3 * H), b_hh.reshape(1, 3 * H),
        Wh1, bh1.reshape(1, H // 2), Wh2p, bh2p)

    return pred[:, :TOUT].transpose(1, 0).reshape(B, TOUT, N)


# speedup vs baseline: 98.9404x; 1.1854x over previous
import jax
import jax.numpy as jnp
from jax import lax
from jax.experimental import pallas as pl
from jax.experimental.pallas import tpu as pltpu
from jax.experimental.pallas import tpu_sc as plsc

B, T, N = 1, 8, 10000
FD, FS, FE = 8, 16, 4
H, HEADS, TOUT = 64, 4, 6
C = H // HEADS
E = 160000

NCORE, NSUB = 2, 16
CHUNK = 128
NCHUNK = 80
EPT = NCHUNK * CHUNK
E_PAD = NSUB * EPT
N_PAD = 10112
RPT = N_PAD // NSUB
ROW = 80
TPC = T // NCORE
XPAD = 4
XPA = H + HEADS + XPAD
NBLK = 2000
ERB = 256
P0R = (E_PAD // CHUNK) // NCORE // NSUB
NBLK_B = 400
EBLK = 2048
TOUT_PAD = 8

_HI = jax.lax.Precision.HIGHEST


def _a0_body(na_ref, w1s_ref, b1_ref, hna_ref):
    hna_ref[...] = (jnp.dot(na_ref[...], w1s_ref[...], precision=_HI)
                    + b1_ref[0])


def _tc_a0(node_attr, w1s, b1r):
    return pl.pallas_call(
        _a0_body,
        grid=(N // NBLK,),
        in_specs=[
            pl.BlockSpec((NBLK, FS), lambda i: (i, 0)),
            pl.BlockSpec((FS, H), lambda i: (0, 0)),
            pl.BlockSpec((1, H), lambda i: (0, 0)),
        ],
        out_specs=pl.BlockSpec((NBLK, H), lambda i: (i, 0)),
        out_shape=jax.ShapeDtypeStruct((N, H), jnp.float32),
    )(node_attr, w1s, b1r)


def _a1_body(x_ref, hna_ref, w1x_ref, wg_ref, asrc_ref,
             adst_ref, hf_ref, xp_ref, als_ref, ald_ref, xpa_ref, ald8_ref):
    x = x_ref[0]
    h = jnp.dot(x, w1x_ref[...], precision=_HI) + hna_ref[...]
    h = jnp.maximum(h, 0.0)
    xp = jnp.dot(h, wg_ref[...], precision=_HI)
    hf_ref[0] = h
    xp_ref[0] = xp
    als = jnp.dot(xp, asrc_ref[...], precision=_HI)
    ald = jnp.dot(xp, adst_ref[...], precision=_HI)
    als_ref[0] = als
    ald_ref[0] = ald
    z4 = jnp.zeros((NBLK, XPAD), jnp.float32)
    xpa_ref[...] = jnp.concatenate([xp, als, z4], axis=1)
    ald8_ref[...] = jnp.concatenate([ald, z4], axis=1)


NB_T = N // NBLK


def _tc_a1(x_seq3, hna, w1x, Wg, Asrc, Adst):
    grid = (T, NB_T)
    out_shape = (
        jax.ShapeDtypeStruct((T, N, H), jnp.float32),
        jax.ShapeDtypeStruct((T, N, H), jnp.float32),
        jax.ShapeDtypeStruct((T, N, HEADS), jnp.float32),
        jax.ShapeDtypeStruct((T, N, HEADS), jnp.float32),
        jax.ShapeDtypeStruct((T * N, XPA), jnp.float32),
        jax.ShapeDtypeStruct((T * N + 128, 2 * HEADS), jnp.float32),
    )
    return pl.pallas_call(
        _a1_body,
        grid=grid,
        in_specs=[
            pl.BlockSpec((1, NBLK, FD), lambda t, i: (t, i, 0)),
            pl.BlockSpec((NBLK, H), lambda t, i: (i, 0)),
            pl.BlockSpec((FD, H), lambda t, i: (0, 0)),
            pl.BlockSpec((H, H), lambda t, i: (0, 0)),
            pl.BlockSpec((H, HEADS), lambda t, i: (0, 0)),
            pl.BlockSpec((H, HEADS), lambda t, i: (0, 0)),
        ],
        out_specs=(
            pl.BlockSpec((1, NBLK, H), lambda t, i: (t, i, 0)),
            pl.BlockSpec((1, NBLK, H), lambda t, i: (t, i, 0)),
            pl.BlockSpec((1, NBLK, HEADS), lambda t, i: (t, i, 0)),
            pl.BlockSpec((1, NBLK, HEADS), lambda t, i: (t, i, 0)),
            pl.BlockSpec((NBLK, XPA), lambda t, i: (t * NB_T + i, 0)),
            pl.BlockSpec((NBLK, 2 * HEADS), lambda t, i: (t * NB_T + i, 0)),
        ),
        out_shape=out_shape,
    )(x_seq3, hna, w1x, Wg, Asrc, Adst)


def _a2_body(ea_ref, m_ref, b_ref, out_ref):
    out_ref[...] = (
        jnp.dot(ea_ref[...], m_ref[...], precision=_HI) + b_ref[0]
    )


def _tc_a2(ea512, M512, b512):
    nrow = E_PAD // CHUNK
    return pl.pallas_call(
        _a2_body,
        grid=(nrow // ERB,),
        in_specs=[
            pl.BlockSpec((ERB, CHUNK * FE), lambda i: (i, 0)),
            pl.BlockSpec((CHUNK * FE, CHUNK * HEADS), lambda i: (0, 0)),
            pl.BlockSpec((1, CHUNK * HEADS), lambda i: (0, 0)),
        ],
        out_specs=pl.BlockSpec((ERB, CHUNK * HEADS), lambda i: (i, 0)),
        out_shape=jax.ShapeDtypeStruct((nrow, CHUNK * HEADS), jnp.float32),
    )(ea512, M512, b512)


def _sc_body(src_h, dst_h, ae_h, xpa_h, ald_h,
             nd_out,
             src_t, dst_t, idx_t, idx2_t,
             ae0, ae1, xg0, xg1, adg0, adg1, msg,
             zrow, nd_sp, se0, se1, sx0, sx1, sa0, sa1):
    cid = lax.axis_index("c")
    sid = lax.axis_index("s")
    rbase = sid * NCHUNK
    zeros16 = jnp.zeros((16,), jnp.float32)
    iota16 = lax.iota(jnp.int32, 16)
    bufs = ((ae0, xg0, adg0, se0, sx0, sa0),
            (ae1, xg1, adg1, se1, sx1, sa1))

    def _zrow_init(i, carry):
        r = i // 5
        c = (i % 5) * 16
        zrow[r, pl.ds(c, 16)] = zeros16
        return carry
    lax.fori_loop(0, 64 * 5, _zrow_init, 0)

    def _msg_init(k, carry):
        for c in range(ROW // 16):
            msg[k, pl.ds(c * 16, 16)] = zeros16
        return carry
    lax.fori_loop(0, CHUNK, _msg_init, 0)

    pltpu.sync_copy(src_h.at[pl.ds(rbase, NCHUNK)], src_t)
    pltpu.sync_copy(dst_h.at[pl.ds(rbase, NCHUNK)], dst_t)

    def _zero_nd_slice():
        for k in range(10):
            sz = 64 if k < 9 else RPT - 9 * 64
            pltpu.sync_copy(zrow.at[pl.ds(0, sz)],
                            nd_sp.at[pl.ds(sid * RPT + k * 64, sz)])

    def _start(j, b):
        ae_b, xg_b, adg_b, se_b, sx_b, sa_b = bufs[b]
        pltpu.async_copy(ae_h.at[rbase + j], ae_b, se_b)
        pltpu.async_copy(xpa_h.at[idx_t.at[j]], xg_b, sx_b)
        pltpu.async_copy(ald_h.at[idx2_t.at[j]], adg_b, sa_b)

    def _wait(j, b):
        ae_b, xg_b, adg_b, se_b, sx_b, sa_b = bufs[b]
        pltpu.make_async_copy(ae_h.at[rbase + j], ae_b, se_b).wait()
        pltpu.make_async_copy(xpa_h.at[idx_t.at[j]], xg_b, sx_b).wait()
        pltpu.make_async_copy(ald_h.at[idx2_t.at[j]], adg_b, sa_b).wait()

    lane_eq = [iota16 == h for h in range(HEADS)]
    zv = jnp.zeros((16,), jnp.float32)

    def _compute(j, b):
        ae_b, xg_b, adg_b = bufs[b][0], bufs[b][1], bufs[b][2]

        def _group(g, c3):
            k16 = iota16 + g * 16
            exvecs = []
            for h in range(HEADS):
                hv = jnp.full((16,), h, jnp.int32)
                a = (plsc.load_gather(xg_b, [k16, hv + H])
                     + plsc.load_gather(adg_b, [k16, hv])
                     + plsc.load_gather(ae_b, [k16 * 4 + h]))
                a = jnp.where(a >= 0.0, a, 0.2 * a)
                exvecs.append(jnp.exp(a))
            for i in range(16):
                k = g * 16 + i
                exv = zv
                for h in range(HEADS):
                    sf = jnp.full((16,), exvecs[h][i])
                    msg[k, pl.ds(h * C, 16)] = xg_b[k, pl.ds(h * C, 16)] * sf
                    exv = jnp.where(lane_eq[h], sf, exv)
                msg[k, pl.ds(H, 16)] = exv
            return c3
        lax.fori_loop(0, 8, _group, 0)
        pltpu.sync_copy(msg, nd_sp.at[dst_t.at[j]], add=True)

    def _t_step(jt, carry):
        t = cid * TPC + jt
        tN = t * N
        _zero_nd_slice()

        def _idx(i, c):
            r = i // 8
            c16 = (i % 8) * 16
            idx_t[r, pl.ds(c16, 16)] = src_t[r, pl.ds(c16, 16)] + tN
            idx2_t[r, pl.ds(c16, 16)] = dst_t[r, pl.ds(c16, 16)] + tN
            return c
        lax.fori_loop(0, NCHUNK * 8, _idx, 0)
        plsc.subcore_barrier()

        _start(0, 0)

        def _pair(g, carry2):
            for b in range(2):
                j = 2 * g + b

                @pl.when(j + 1 < NCHUNK)
                def _pre():
                    _start(j + 1, 1 - b)
                _wait(j, b)
                _compute(j, b)
            return carry2
        lax.fori_loop(0, NCHUNK // 2, _pair, 0)
        plsc.subcore_barrier()
        for k in range(10):
            sz = 64 if k < 9 else RPT - 9 * 64
            r0 = sid * RPT + k * 64
            pltpu.sync_copy(nd_sp.at[pl.ds(r0, sz)],
                            nd_out.at[t, pl.ds(r0, sz)])
        plsc.subcore_barrier()
        return carry
    lax.fori_loop(0, TPC, _t_step, 0)


def _sc_gat(src2, dst2, ae2, xpa_f, ald_f):
    mesh = plsc.VectorSubcoreMesh(core_axis_name="c", subcore_axis_name="s",
                                  num_cores=NCORE, num_subcores=NSUB)
    kfn = pl.kernel(
        _sc_body,
        out_type=jax.ShapeDtypeStruct((T, N_PAD, ROW), jnp.float32),
        mesh=mesh,
        scratch_types=[
            pltpu.VMEM((NCHUNK, CHUNK), jnp.int32),
            pltpu.VMEM((NCHUNK, CHUNK), jnp.int32),
            pltpu.VMEM((NCHUNK, CHUNK), jnp.int32),
            pltpu.VMEM((NCHUNK, CHUNK), jnp.int32),
            pltpu.VMEM((CHUNK * 4,), jnp.float32),
            pltpu.VMEM((CHUNK * 4,), jnp.float32),
            pltpu.VMEM((CHUNK, XPA), jnp.float32),
            pltpu.VMEM((CHUNK, XPA), jnp.float32),
            pltpu.VMEM((CHUNK, 2 * HEADS), jnp.float32),
            pltpu.VMEM((CHUNK, 2 * HEADS), jnp.float32),
            pltpu.VMEM((CHUNK, ROW), jnp.float32),
            pltpu.VMEM((64, ROW), jnp.float32),
            pltpu.VMEM_SHARED((N_PAD, ROW), jnp.float32),
            pltpu.SemaphoreType.DMA,
            pltpu.SemaphoreType.DMA,
            pltpu.SemaphoreType.DMA,
            pltpu.SemaphoreType.DMA,
            pltpu.SemaphoreType.DMA,
            pltpu.SemaphoreType.DMA,
        ],
        compiler_params=pltpu.CompilerParams(needs_layout_passes=False,
                                             use_tc_tiling_on_sc=False),
    )
    return kfn(src2, dst2, ae2, xpa_f, ald_f)


def _p0_body(dst_h, ae_h, deg_out, dst_c, ae_c, msg, zrow, nd_sp):
    cid = lax.axis_index("c")
    sid = lax.axis_index("s")
    rbase = (cid * NSUB + sid) * P0R
    zeros16 = jnp.zeros((16,), jnp.float32)
    iota16 = lax.iota(jnp.int32, 16)

    def _zrow_init(i, carry):
        r = i // 5
        c = (i % 5) * 16
        zrow[r, pl.ds(c, 16)] = zeros16
        return carry
    lax.fori_loop(0, 64 * 5, _zrow_init, 0)

    def _msg_init(k, carry):
        for c in range(ROW // 16):
            msg[k, pl.ds(c * 16, 16)] = zeros16
        return carry
    lax.fori_loop(0, CHUNK, _msg_init, 0)

    for k in range(10):
        sz = 64 if k < 9 else RPT - 9 * 64
        pltpu.sync_copy(zrow.at[pl.ds(0, sz)],
                        nd_sp.at[pl.ds(sid * RPT + k * 64, sz)])
    plsc.subcore_barrier()

    lane_eq5 = [iota16 == h for h in range(5)]
    base_row = jnp.where(lane_eq5[0], 1.0, 0.0)

    def _p0_chunk(j, carry):
        pltpu.sync_copy(dst_h.at[rbase + j], dst_c)
        pltpu.sync_copy(ae_h.at[rbase + j], ae_c)

        def _p0_group(g, c2):
            k16 = iota16 + g * 16
            aevecs = [plsc.load_gather(ae_c, [k16 * 4 + h])
                      for h in range(HEADS)]
            for i in range(16):
                k = g * 16 + i
                v = base_row
                for h in range(HEADS):
                    v = jnp.where(lane_eq5[1 + h],
                                  jnp.full((16,), aevecs[h][i]), v)
                msg[k, pl.ds(0, 16)] = v
            return c2
        lax.fori_loop(0, 8, _p0_group, 0)
        pltpu.sync_copy(msg, nd_sp.at[dst_c], add=True)
        return carry
    lax.fori_loop(0, P0R, _p0_chunk, 0)
    plsc.subcore_barrier()
    for k in range(10):
        sz = 64 if k < 9 else RPT - 9 * 64
        r0 = sid * RPT + k * 64
        pltpu.sync_copy(nd_sp.at[pl.ds(r0, sz)],
                        deg_out.at[cid, pl.ds(r0, sz)])


def _sc_p0(dst2, ae2):
    mesh = plsc.VectorSubcoreMesh(core_axis_name="c", subcore_axis_name="s",
                                  num_cores=NCORE, num_subcores=NSUB)
    kfn = pl.kernel(
        _p0_body,
        out_type=jax.ShapeDtypeStruct((NCORE, N_PAD, ROW), jnp.float32),
        mesh=mesh,
        scratch_types=[
            pltpu.VMEM((CHUNK,), jnp.int32),
            pltpu.VMEM((CHUNK * 4,), jnp.float32),
            pltpu.VMEM((CHUNK, ROW), jnp.float32),
            pltpu.VMEM((64, ROW), jnp.float32),
            pltpu.VMEM_SHARED((N_PAD, ROW), jnp.float32),
        ],
        compiler_params=pltpu.CompilerParams(needs_layout_passes=False,
                                             use_tc_tiling_on_sc=False),
    )
    return kfn(dst2, ae2)


def _b_body(nd_ref, hf_ref, xp_ref, als_ref, ald_ref, deg_ref, exp_ref,
            gb_ref, lng_ref, lnb_ref, wih_ref, whh_ref, bih_ref, bhh_ref,
            wh1_ref, bh1_ref, wh2_ref, bh2_ref, out_ref):
    nd = nd_ref[...]
    num = nd[:, :, :H]
    den4 = nd[:, :, H:H + HEADS]
    degs = deg_ref[0] + deg_ref[1]
    deg = jnp.maximum(degs[:, 0], 1.0)
    ael4 = degs[:, 1:1 + HEADS] / deg[:, None]
    al = als_ref[...] + ald_ref[...] + ael4[None]
    al = jnp.where(al >= 0.0, al, 0.2 * al)
    exl = jnp.exp(al)
    expand = exp_ref[...]
    exl64 = jnp.dot(exl.reshape(T * NBLK_B, HEADS), expand,
                    precision=_HI).reshape(T, NBLK_B, H)
    den64 = jnp.dot(den4.reshape(T * NBLK_B, HEADS), expand,
                    precision=_HI).reshape(T, NBLK_B, H)
    xp = xp_ref[...]
    agg = (num + exl64 * xp) / (den64 + exl64 + 1e-16)
    y = agg + gb_ref[0] + hf_ref[...]
    mu = jnp.mean(y, axis=-1, keepdims=True)
    var = jnp.mean((y - mu) ** 2, axis=-1, keepdims=True)
    y = (y - mu) / jnp.sqrt(var + 1e-5) * lng_ref[0] + lnb_ref[0]

    wih = wih_ref[...]
    whh = whh_ref[...]
    bih = bih_ref[0]
    bhh = bhh_ref[0]
    hst = jnp.zeros((NBLK_B, H), jnp.float32)
    dn = (((1,), (1,)), ((), ()))
    for t in range(T):
        x_t = y[t]
        gi = lax.dot_general(x_t, wih, dn, precision=_HI) + bih
        gh = lax.dot_general(hst, whh, dn, precision=_HI) + bhh
        r = jax.nn.sigmoid(gi[:, :H] + gh[:, :H])
        z = jax.nn.sigmoid(gi[:, H:2 * H] + gh[:, H:2 * H])
        n = jnp.tanh(gi[:, 2 * H:] + r * gh[:, 2 * H:])
        hst = (1.0 - z) * n + z * hst
    hid = jnp.maximum(jnp.dot(hst, wh1_ref[...], precision=_HI) + bh1_ref[0],
                      0.0)
    out_ref[...] = jnp.dot(hid, wh2_ref[...], precision=_HI) + bh2_ref[0]


def _tc_b(nd, hflat, xproj, als, ald, degsum, expand, gb, lng, lnb,
          wih, whh, bih, bhh, Wh1, bh1, Wh2p, bh2p):
    def full(shape):
        return pl.BlockSpec(shape, lambda i, _s=shape: tuple(0 for _ in _s))
    return pl.pallas_call(
        _b_body,
        grid=(N // NBLK_B,),
        in_specs=[
            pl.BlockSpec((T, NBLK_B, ROW), lambda i: (0, i, 0)),
            pl.BlockSpec((T, NBLK_B, H), lambda i: (0, i, 0)),
            pl.BlockSpec((T, NBLK_B, H), lambda i: (0, i, 0)),
            pl.BlockSpec((T, NBLK_B, HEADS), lambda i: (0, i, 0)),
            pl.BlockSpec((T, NBLK_B, HEADS), lambda i: (0, i, 0)),
            pl.BlockSpec((NCORE, NBLK_B, ROW), lambda i: (0, i, 0)),
            full((HEADS, H)),
            full((1, H)),
            full((1, H)),
            full((1, H)),
            full((3 * H, H)),
            full((3 * H, H)),
            full((1, 3 * H)),
            full((1, 3 * H)),
            full((H, H // 2)),
            full((1, H // 2)),
            full((H // 2, TOUT_PAD)),
            full((1, TOUT_PAD)),
        ],
        out_specs=pl.BlockSpec((NBLK_B, TOUT_PAD), lambda i: (i, 0)),
        out_shape=jax.ShapeDtypeStruct((N, TOUT_PAD), jnp.float32),
    )(nd, hflat, xproj, als, ald, degsum, expand, gb, lng, lnb,
      wih, whh, bih, bhh, Wh1, bh1, Wh2p, bh2p)


def kernel(x_seq, node_attr, edge_index, edge_attr, W1, b1, We, be, Wg,
           att_src, att_dst, Wle, att_edge, gat_bias, ln_g, ln_b, w_ih,
           w_hh, b_ih, b_hh, Wh1, bh1, Wh2, bh2):
    f32 = jnp.float32
    Ve = jnp.einsum("dhc,hc->dh", Wle.reshape(H, HEADS, C), att_edge[0])
    WeVe = We @ Ve
    M512 = jnp.kron(jnp.eye(CHUNK, dtype=f32), WeVe)
    b512 = jnp.tile((be @ Ve).reshape(1, HEADS), (1, CHUNK))
    eye = jnp.eye(HEADS, dtype=f32)
    Asrc = (att_src[0][:, :, None] * eye[:, None, :]).reshape(H, HEADS)
    Adst = (att_dst[0][:, :, None] * eye[:, None, :]).reshape(H, HEADS)
    expand = jnp.kron(eye, jnp.ones((1, C), f32))
    w1x = W1[:FD]
    w1s = W1[FD:]
    Wh2p = jnp.pad(Wh2, ((0, 0), (0, TOUT_PAD - TOUT)))
    bh2p = jnp.pad(bh2, (0, TOUT_PAD - TOUT)).reshape(1, TOUT_PAD)

    src = edge_index[0]
    dst = edge_index[1]
    npad = E_PAD - E
    src2 = jnp.concatenate([src, jnp.zeros((npad,), jnp.int32)]
                           ).reshape(-1, CHUNK)
    dst2 = jnp.concatenate([dst, jnp.full((npad,), N, jnp.int32)]
                           ).reshape(-1, CHUNK)
    ea512 = jnp.concatenate(
        [edge_attr.reshape(E * FE // (CHUNK * FE), CHUNK * FE),
         jnp.zeros((npad // CHUNK, CHUNK * FE), f32)])

    x_seq3 = x_seq.reshape(T, N, FD)

    aedge2 = _tc_a2(ea512, M512, b512)
    degsum = _sc_p0(dst2, aedge2)
    hna = _tc_a0(node_attr, w1s, b1.reshape(1, H))
    hflat, xproj, als, ald, xpa2, ald8 = _tc_a1(
        x_seq3, hna, w1x, Wg, Asrc, Adst)

    nd = _sc_gat(src2, dst2, aedge2, xpa2, ald8)

    pred = _tc_b(
        nd, hflat, xproj, als, ald, degsum, expand,
        gat_bias.reshape(1, H), ln_g.reshape(1, H), ln_b.reshape(1, H),
        w_ih, w_hh, b_ih.reshape(1, 3 * H), b_hh.reshape(1, 3 * H),
        Wh1, bh1.reshape(1, H // 2), Wh2p, bh2p)

    return pred[:, :TOUT].transpose(1, 0).reshape(B, TOUT, N)
